# baseline (speedup 1.0000x reference)
import functools

import jax
import jax.numpy as jnp
from jax import lax
from jax.experimental import pallas as pl
from jax.experimental.pallas import tpu as pltpu
from jax.experimental.pallas import tpu_sc as plsc

N = 16384
C = 1000
NC = 2
NS = 16
NW = NC * NS
B = N // NW
NCHUNK = 4
CHUNK = B // NCHUNK
L = 16


@functools.partial(
    pl.kernel,
    mesh=plsc.VectorSubcoreMesh(core_axis_name="c", subcore_axis_name="s"),
    out_type=jax.ShapeDtypeStruct((NW, L), jnp.float32),
    scratch_types=[
        pltpu.VMEM((NCHUNK, CHUNK), jnp.int32),
        pltpu.VMEM((NCHUNK, CHUNK), jnp.float32),
        pltpu.VMEM((B,), jnp.int32),
        pltpu.VMEM((B,), jnp.float32),
        pltpu.VMEM((L,), jnp.float32),
        pltpu.SemaphoreType.DMA,
    ],
)
def _gan_loss_sc(prob_hbm, target_hbm, reward_hbm, out_hbm,
                 idx_v, vals_v, tgt_v, rew_v, part_v, sem):
    wid = lax.axis_index("s") * NC + lax.axis_index("c")
    base = wid * B

    pltpu.sync_copy(target_hbm.at[pl.ds(base, B)], tgt_v)
    pltpu.sync_copy(reward_hbm.at[pl.ds(base, B)], rew_v)

    for r in range(NCHUNK):
        for k in range(CHUNK // L):
            off = r * CHUNK + k * L
            t = tgt_v[pl.ds(off, L)]
            n = (base + off) + lax.iota(jnp.int32, L)
            idx = (((t >> 3) << 17) + ((n >> 7) << 10)
                   + ((t & 7) << 7) + (n & 127))
            idx_v[r, pl.ds(k * L, L)] = idx

    copies = [
        pltpu.async_copy(prob_hbm.at[idx_v.at[r]], vals_v.at[r], sem)
        for r in range(NCHUNK)
    ]
    for cp in copies:
        cp.wait()

    acc = jnp.zeros((L,), jnp.float32)
    for r in range(NCHUNK):
        for k in range(CHUNK // L):
            off = r * CHUNK + k * L
            acc = acc + vals_v[r, pl.ds(k * L, L)] * rew_v[pl.ds(off, L)]

    part_v[...] = acc
    pltpu.sync_copy(part_v, out_hbm.at[wid])


def _reduce_tc_body(parts_ref, out_ref):
    out_ref[0, 0] = -jnp.sum(parts_ref[...])


_reduce_tc = pl.pallas_call(
    _reduce_tc_body,
    out_shape=jax.ShapeDtypeStruct((1, 1), jnp.float32),
    in_specs=[pl.BlockSpec(memory_space=pltpu.VMEM)],
    out_specs=pl.BlockSpec(memory_space=pltpu.SMEM),
)


def kernel(prob, target, reward):
    flat = (prob.T.reshape(C // 8, 8, N // 128, 128)
            .transpose(0, 2, 1, 3)
            .reshape(N * C))
    parts = _gan_loss_sc(flat, target, reward)
    return jnp.reshape(_reduce_tc(parts), ())


# speedup vs baseline: 1.0459x; 1.0459x over previous
import functools

import jax
import jax.numpy as jnp
from jax import lax
from jax.experimental import pallas as pl
from jax.experimental.pallas import tpu as pltpu
from jax.experimental.pallas import tpu_sc as plsc

N = 16384
C = 1000
NT = 16
B = N // NT
NCHUNK = 8
CHUNK = B // NCHUNK
L = 16


@functools.partial(
    pl.kernel,
    mesh=plsc.VectorSubcoreMesh(core_axis_name="c", subcore_axis_name="s",
                                num_cores=1),
    out_type=jax.ShapeDtypeStruct((NT, L), jnp.float32),
    scratch_types=[
        pltpu.VMEM((NCHUNK, CHUNK), jnp.int32),
        pltpu.VMEM((NCHUNK, CHUNK), jnp.float32),
        pltpu.VMEM((B,), jnp.int32),
        pltpu.VMEM((B,), jnp.float32),
        pltpu.VMEM((L,), jnp.float32),
        pltpu.SemaphoreType.DMA,
    ],
)
def _gan_loss_sc(prob_hbm, target_hbm, reward_hbm, out_hbm,
                 idx_v, vals_v, tgt_v, rew_v, part_v, sem):
    sid = lax.axis_index("s")
    base = sid * B

    cp_t = pltpu.async_copy(target_hbm.at[pl.ds(base, B)], tgt_v, sem)
    cp_r = pltpu.async_copy(reward_hbm.at[pl.ds(base, B)], rew_v, sem)
    cp_t.wait()
    cp_r.wait()

    for r in range(NCHUNK):
        for k in range(CHUNK // L):
            off = r * CHUNK + k * L
            t = tgt_v[pl.ds(off, L)]
            n = (base + off) + lax.iota(jnp.int32, L)
            idx = (((t >> 3) << 17) + ((n >> 7) << 10)
                   + ((t & 7) << 7) + (n & 127))
            idx_v[r, pl.ds(k * L, L)] = idx

    copies = [
        pltpu.async_copy(prob_hbm.at[idx_v.at[r]], vals_v.at[r], sem)
        for r in range(NCHUNK)
    ]
    for cp in copies:
        cp.wait()

    acc = jnp.zeros((L,), jnp.float32)
    for r in range(NCHUNK):
        for k in range(CHUNK // L):
            off = r * CHUNK + k * L
            acc = acc + vals_v[r, pl.ds(k * L, L)] * rew_v[pl.ds(off, L)]

    part_v[...] = acc
    pltpu.sync_copy(part_v, out_hbm.at[sid])


def _reduce_tc_body(parts_hbm, out_ref, parts_v, sem):
    pltpu.make_async_copy(parts_hbm, parts_v, sem).start()
    pltpu.make_async_copy(parts_hbm, parts_v, sem).wait()
    out_ref[0, 0] = -jnp.sum(parts_v[...])


_reduce_tc = pl.pallas_call(
    _reduce_tc_body,
    out_shape=jax.ShapeDtypeStruct((1, 1), jnp.float32),
    in_specs=[pl.BlockSpec(memory_space=pl.ANY)],
    out_specs=pl.BlockSpec(memory_space=pltpu.SMEM),
    scratch_shapes=[
        pltpu.VMEM((NT, L), jnp.float32),
        pltpu.SemaphoreType.DMA,
    ],
)


def kernel(prob, target, reward):
    flat = (prob.T.reshape(C // 8, 8, N // 128, 128)
            .transpose(0, 2, 1, 3)
            .reshape(N * C))
    parts = _gan_loss_sc(flat, target, reward)
    return jnp.reshape(_reduce_tc(parts), ())
